# baseline (speedup 1.0000x reference)
import jax
import jax.numpy as jnp
from jax.experimental import pallas as pl


def _mm_block(support_ref, adj_ref, out_ref):
    out_ref[...] = jnp.dot(
        adj_ref[...].astype(jnp.bfloat16),
        support_ref[...].astype(jnp.bfloat16),
        preferred_element_type=jnp.float32,
    )


def kernel(support, adj):
    n, d = support.shape
    bm = 128
    grid_m = -(-n // bm)
    return pl.pallas_call(
        _mm_block,
        grid=(grid_m,),
        in_specs=[
            pl.BlockSpec((n, d), lambda i: (0, 0)),
            pl.BlockSpec((bm, n), lambda i: (i, 0)),
        ],
        out_specs=pl.BlockSpec((bm, d), lambda i: (i, 0)),
        out_shape=jax.ShapeDtypeStruct((n, d), jnp.float32),
    )(support, adj)


# speedup vs baseline: 1.0930x; 1.0930x over previous
import jax
import jax.numpy as jnp
from jax.experimental import pallas as pl
from jax.experimental.pallas import tpu as pltpu

_BM = 200
_NBUF = 4


def _mm_manual(support_ref, adj_hbm, out_ref, buf, sem):
    i = pl.program_id(0)
    nsteps = pl.num_programs(0)

    def _copy(blk, slot):
        return pltpu.make_async_copy(
            adj_hbm.at[pl.ds(blk * _BM, _BM), :], buf.at[slot], sem.at[slot]
        )

    @pl.when(i == 0)
    def _():
        for b in range(_NBUF):
            _copy(b, b).start()

    @pl.when((i > 0) & (i + _NBUF - 1 < nsteps))
    def _():
        blk = i + _NBUF - 1
        _copy(blk, jax.lax.rem(blk, _NBUF)).start()

    slot = jax.lax.rem(i, _NBUF)
    _copy(i, slot).wait()
    out_ref[...] = jnp.dot(
        buf[slot], support_ref[...], preferred_element_type=jnp.float32
    )


def kernel(support, adj):
    n, d = support.shape
    assert n % _BM == 0 and n // _BM >= _NBUF
    return pl.pallas_call(
        _mm_manual,
        grid=(n // _BM,),
        in_specs=[
            pl.BlockSpec((n, d), lambda i: (0, 0)),
            pl.BlockSpec(memory_space=pl.ANY),
        ],
        out_specs=pl.BlockSpec((_BM, d), lambda i: (i, 0)),
        out_shape=jax.ShapeDtypeStruct((n, d), jnp.float32),
        scratch_shapes=[
            pltpu.VMEM((_NBUF, _BM, n), jnp.float32),
            pltpu.SemaphoreType.DMA((_NBUF,)),
        ],
    )(support, adj)


# speedup vs baseline: 1.1133x; 1.0185x over previous
import jax
import jax.numpy as jnp
from jax.experimental import pallas as pl
from jax.experimental.pallas import tpu as pltpu


def _mm_block(support_ref, adj_ref, out_ref):
    out_ref[...] = jnp.dot(
        adj_ref[...], support_ref[...], preferred_element_type=jnp.float32
    )


def kernel(support, adj):
    n, d = support.shape
    bm = 200
    grid_m = -(-n // bm)
    return pl.pallas_call(
        _mm_block,
        grid=(grid_m,),
        in_specs=[
            pl.BlockSpec((n, d), lambda i: (0, 0)),
            pl.BlockSpec((bm, n), lambda i: (i, 0)),
        ],
        out_specs=pl.BlockSpec((bm, d), lambda i: (i, 0)),
        out_shape=jax.ShapeDtypeStruct((n, d), jnp.float32),
        compiler_params=pltpu.CompilerParams(
            dimension_semantics=("parallel",),
        ),
    )(support, adj)


# speedup vs baseline: 1.1323x; 1.0171x over previous
import jax
import jax.numpy as jnp
from jax.experimental import pallas as pl
from jax.experimental.pallas import tpu as pltpu


def _mm_block(support_ref, adj_ref, out_ref):
    out_ref[...] = jnp.dot(
        adj_ref[...], support_ref[...], preferred_element_type=jnp.float32
    )


def kernel(support, adj):
    n, d = support.shape
    bm = 240
    grid_m = -(-n // bm)
    return pl.pallas_call(
        _mm_block,
        grid=(grid_m,),
        in_specs=[
            pl.BlockSpec((n, d), lambda i: (0, 0)),
            pl.BlockSpec((bm, n), lambda i: (i, 0)),
        ],
        out_specs=pl.BlockSpec((bm, d), lambda i: (i, 0)),
        out_shape=jax.ShapeDtypeStruct((n, d), jnp.float32),
        compiler_params=pltpu.CompilerParams(
            dimension_semantics=("parallel",),
        ),
    )(support, adj)
